# Initial kernel scaffold; baseline (speedup 1.0000x reference)
import functools

import jax
import jax.numpy as jnp
from jax import lax
from jax.experimental import pallas as pl
from jax.experimental.pallas import tpu as pltpu
from jax.experimental.pallas import tpu_sc as plsc

EMBED = 64
BATCH = 16384
MAXTXT = 50
B_TOTAL = BATCH * MAXTXT
NW = 32
ROWS_PER_W = B_TOTAL // NW
IDX_MINOR = 128
CHUNK = 512
NDMA = CHUNK // IDX_MINOR
NITER = ROWS_PER_W // CHUNK


@functools.partial(
    pl.kernel,
    mesh=plsc.VectorSubcoreMesh(core_axis_name="c", subcore_axis_name="s"),
    out_type=jax.ShapeDtypeStruct((B_TOTAL, EMBED), jnp.float32),
    scratch_types=[
        pltpu.VMEM((NDMA, IDX_MINOR), jnp.int32),
        pltpu.VMEM((CHUNK, EMBED), jnp.float32),
        pltpu.SemaphoreType.DMA,
    ],
)
def _gather_kernel(table_hbm, idx_hbm, out_hbm, idx_v, rows_v, sem):
    wid = lax.axis_index("s") * 2 + lax.axis_index("c")
    base = wid * ROWS_PER_W

    def body(g, carry):
        off = base + g * CHUNK
        row = off // IDX_MINOR
        pltpu.sync_copy(idx_hbm.at[pl.ds(row, NDMA)], idx_v)
        handles = []
        for j in range(NDMA):
            h = pltpu.async_copy(
                table_hbm.at[idx_v.at[j]],
                rows_v.at[pl.ds(j * IDX_MINOR, IDX_MINOR)],
                sem,
            )
            handles.append(h)
        for h in handles:
            h.wait()
        pltpu.sync_copy(rows_v, out_hbm.at[pl.ds(off, CHUNK)])
        return carry

    lax.fori_loop(0, NITER, body, 0)


def kernel(index, table):
    idx2d = index.reshape(B_TOTAL // IDX_MINOR, IDX_MINOR)
    out = _gather_kernel(table, idx2d)
    return out.reshape(BATCH, MAXTXT, EMBED)


# speedup vs baseline: 1.8443x; 1.8443x over previous
import functools

import jax
import jax.numpy as jnp
from jax import lax
from jax.experimental import pallas as pl
from jax.experimental.pallas import tpu as pltpu
from jax.experimental.pallas import tpu_sc as plsc

EMBED = 64
BATCH = 16384
MAXTXT = 50
B_TOTAL = BATCH * MAXTXT
NW = 32
ROWS_PER_W = B_TOTAL // NW
IDX_MINOR = 128
CHUNK = 1024
NDMA = CHUNK // IDX_MINOR
NITER = ROWS_PER_W // CHUNK


@functools.partial(
    pl.kernel,
    mesh=plsc.VectorSubcoreMesh(core_axis_name="c", subcore_axis_name="s"),
    out_type=jax.ShapeDtypeStruct((B_TOTAL, EMBED), jnp.float32),
    scratch_types=[
        pltpu.VMEM((NDMA, IDX_MINOR), jnp.int32),
        pltpu.VMEM((CHUNK, EMBED), jnp.float32),
        pltpu.SemaphoreType.DMA,
    ],
    compiler_params=pltpu.CompilerParams(use_tc_tiling_on_sc=False),
)
def _gather_kernel(table_hbm, idx_hbm, out_hbm, idx_v, rows_v, sem):
    wid = lax.axis_index("s") * 2 + lax.axis_index("c")
    base = wid * ROWS_PER_W

    def body(g, carry):
        off = pl.multiple_of(base + g * CHUNK, CHUNK)
        row = pl.multiple_of(off // IDX_MINOR, NDMA)
        pltpu.sync_copy(idx_hbm.at[pl.ds(row, NDMA)], idx_v)
        handles = []
        for j in range(NDMA):
            h = pltpu.async_copy(
                table_hbm.at[idx_v.at[j]],
                rows_v.at[pl.ds(j * IDX_MINOR, IDX_MINOR)],
                sem,
            )
            handles.append(h)
        for h in handles:
            h.wait()
        pltpu.sync_copy(rows_v, out_hbm.at[pl.ds(off, CHUNK)])
        return carry

    lax.fori_loop(0, NITER, body, 0)


def kernel(index, table):
    idx2d = index.reshape(B_TOTAL // IDX_MINOR, IDX_MINOR)
    out = _gather_kernel(table, idx2d)
    return out.reshape(BATCH, MAXTXT, EMBED)


# speedup vs baseline: 1.8775x; 1.0180x over previous
import functools

import jax
import jax.numpy as jnp
from jax import lax
from jax.experimental import pallas as pl
from jax.experimental.pallas import tpu as pltpu
from jax.experimental.pallas import tpu_sc as plsc

EMBED = 64
BATCH = 16384
MAXTXT = 50
B_TOTAL = BATCH * MAXTXT
NW = 32
ROWS_PER_W = B_TOTAL // NW
CHUNK = 128
NBUF = 8
NCHUNK = ROWS_PER_W // CHUNK
NROUND = NCHUNK // NBUF


@functools.partial(
    pl.kernel,
    mesh=plsc.VectorSubcoreMesh(core_axis_name="c", subcore_axis_name="s"),
    out_type=jax.ShapeDtypeStruct((B_TOTAL, EMBED), jnp.float32),
    scratch_types=[
        pltpu.VMEM((NCHUNK, CHUNK), jnp.int32),
        pltpu.VMEM((NBUF, CHUNK, EMBED), jnp.float32),
        pltpu.SemaphoreType.DMA((NBUF,)),
        pltpu.SemaphoreType.DMA((NBUF,)),
    ],
    compiler_params=pltpu.CompilerParams(use_tc_tiling_on_sc=False),
)
def _gather_kernel(table_hbm, idx_hbm, out_hbm, idx_v, rows_v, sem_g, sem_o):
    wid = lax.axis_index("s") * 2 + lax.axis_index("c")
    base = wid * ROWS_PER_W
    idx_row0 = wid * NCHUNK

    pltpu.sync_copy(idx_hbm.at[pl.ds(idx_row0, NCHUNK)], idx_v)

    def fire(c, b):
        pltpu.async_copy(table_hbm.at[idx_v.at[c]], rows_v.at[b], sem_g.at[b])

    def wait_gather(b):
        pltpu.make_async_copy(
            table_hbm.at[pl.ds(0, CHUNK)], rows_v.at[b], sem_g.at[b]
        ).wait()

    def store(c, b):
        pltpu.async_copy(
            rows_v.at[b], out_hbm.at[pl.ds(base + c * CHUNK, CHUNK)], sem_o.at[b]
        )

    def wait_store(b):
        pltpu.make_async_copy(
            table_hbm.at[pl.ds(0, CHUNK)], rows_v.at[b], sem_o.at[b]
        ).wait()

    for b in range(4):
        fire(b, b)

    def slot(c, b, do_retire, do_fire):
        wait_gather(b)
        store(c, b)
        b2 = (b + 4) % NBUF
        if do_retire:
            wait_store(b2)
        if do_fire:
            fire(c + 4, b2)

    for b in range(NBUF):
        slot(b, b, do_retire=b >= 4, do_fire=True)

    def round_body(r, carry):
        c0 = r * NBUF
        for b in range(NBUF):
            slot(c0 + b, b, do_retire=True, do_fire=True)
        return carry

    lax.fori_loop(1, NROUND - 1, round_body, 0)

    c0 = (NROUND - 1) * NBUF
    for b in range(NBUF):
        slot(c0 + b, b, do_retire=True, do_fire=b < 4)

    for b in range(4, NBUF):
        wait_store(b)


def kernel(index, table):
    idx2d = index.reshape(B_TOTAL // CHUNK, CHUNK)
    out = _gather_kernel(table, idx2d)
    return out.reshape(BATCH, MAXTXT, EMBED)


# speedup vs baseline: 1.9003x; 1.0121x over previous
import functools

import jax
import jax.numpy as jnp
from jax import lax
from jax.experimental import pallas as pl
from jax.experimental.pallas import tpu as pltpu
from jax.experimental.pallas import tpu_sc as plsc

EMBED = 64
BATCH = 16384
MAXTXT = 50
B_TOTAL = BATCH * MAXTXT
NW = 32
ROWS_PER_W = B_TOTAL // NW
CHUNK = 128
NBUF = 8
NCHUNK = ROWS_PER_W // CHUNK
NROUND = NCHUNK // NBUF


@functools.partial(
    pl.kernel,
    mesh=plsc.VectorSubcoreMesh(core_axis_name="c", subcore_axis_name="s"),
    out_type=jax.ShapeDtypeStruct((B_TOTAL, EMBED), jnp.float32),
    scratch_types=[
        pltpu.VMEM((NCHUNK, CHUNK), jnp.int32),
        pltpu.VMEM((NBUF, CHUNK, EMBED), jnp.float32),
        pltpu.SemaphoreType.DMA((NBUF,)),
        pltpu.SemaphoreType.DMA((NBUF,)),
    ],
    compiler_params=pltpu.CompilerParams(use_tc_tiling_on_sc=False),
)
def _gather_kernel(table_hbm, idx_hbm, out_hbm, idx_v, rows_v, sem_g, sem_o):
    wid = lax.axis_index("s") * 2 + lax.axis_index("c")
    base = wid * ROWS_PER_W
    idx_row0 = wid * NCHUNK

    pltpu.sync_copy(idx_hbm.at[pl.ds(idx_row0, NCHUNK)], idx_v)

    def fire(c, b):
        pltpu.async_copy(table_hbm.at[idx_v.at[c]], rows_v.at[b], sem_g.at[b])

    def wait_gather(b):
        pltpu.make_async_copy(
            table_hbm.at[pl.ds(0, CHUNK)], rows_v.at[b], sem_g.at[b]
        ).wait()

    def store(c, b):
        pltpu.async_copy(
            rows_v.at[b], out_hbm.at[pl.ds(base + c * CHUNK, CHUNK)], sem_o.at[b]
        )

    def wait_store(b):
        pltpu.make_async_copy(
            table_hbm.at[pl.ds(0, CHUNK)], rows_v.at[b], sem_o.at[b]
        ).wait()

    for b in range(4):
        fire(b, b)

    def slot(c, b, do_retire, do_fire):
        wait_gather(b)
        store(c, b)
        b2 = (b + 4) % NBUF
        if do_retire:
            wait_store(b2)
        if do_fire:
            fire(c + 4, b2)

    for b in range(NBUF):
        slot(b, b, do_retire=b >= 4, do_fire=True)

    def round_body(r, carry):
        c0 = r * NBUF
        for b in range(NBUF):
            slot(c0 + b, b, do_retire=True, do_fire=True)
        return carry

    lax.fori_loop(1, NROUND - 1, round_body, 0)

    c0 = (NROUND - 1) * NBUF
    for b in range(NBUF):
        slot(c0 + b, b, do_retire=True, do_fire=b < 4)

    for b in range(4, NBUF):
        wait_store(b)


VOCAB = 1000000
TR_COLS = 2048


def _transpose_body(x_ref, o_ref):
    x = x_ref[...]
    eye = (jax.lax.broadcasted_iota(jnp.int32, (EMBED, EMBED), 0)
           == jax.lax.broadcasted_iota(jnp.int32, (EMBED, EMBED), 1)
           ).astype(jnp.float32)
    y = jax.lax.dot_general(
        x, eye, (((0,), (0,)), ((), ())),
        preferred_element_type=jnp.float32)
    y = y.reshape(TR_COLS // 2, 2, EMBED)
    o_ref[...] = jnp.concatenate([y[:, 0, :], y[:, 1, :]], axis=1)


_transpose_table = pl.pallas_call(
    _transpose_body,
    grid=((VOCAB + TR_COLS - 1) // TR_COLS,),
    in_specs=[pl.BlockSpec((EMBED, TR_COLS), lambda j: (0, j))],
    out_specs=pl.BlockSpec((TR_COLS // 2, 2 * EMBED), lambda j: (j, 0)),
    out_shape=jax.ShapeDtypeStruct((VOCAB // 2, 2 * EMBED), jnp.float32),
)


def kernel(index, table):
    table_rm = _transpose_table(jnp.swapaxes(table, 0, 1))
    idx2d = index.reshape(B_TOTAL // CHUNK, CHUNK)
    out = _gather_kernel(table_rm.reshape(VOCAB, EMBED), idx2d)
    return out.reshape(BATCH, MAXTXT, EMBED)
